# baseline (speedup 1.0000x reference)
import jax
import jax.numpy as jnp
from jax.experimental import pallas as pl

POOL = (2, 2)


def _copy_body(x_ref, o_ref):
    o_ref[...] = x_ref[...]


def kernel(updates, mask):
    B, H, W, C = updates.shape
    ph, pw = POOL
    height, width = H * ph, W * pw
    flat_input_size = H * W * C
    flat_output_size = height * width * C
    mask_i = mask.astype(jnp.int32)
    flat_mask = mask_i.reshape(-1)
    batch_range = jnp.arange(B, dtype=jnp.int32).reshape(-1, 1, 1, 1)
    b = jnp.ones_like(mask_i) * batch_range
    flat_mask = flat_mask + b.reshape(-1) * jnp.int32(flat_input_size)
    flat_updates = updates.reshape(-1)
    flat_output = jnp.zeros((B * flat_output_size,), dtype=updates.dtype)
    flat_output = flat_output.at[flat_mask].add(flat_updates)

    x = flat_output.reshape(4704, 8192)
    out = pl.pallas_call(
        _copy_body,
        out_shape=jax.ShapeDtypeStruct(x.shape, x.dtype),
        grid=(588,),
        in_specs=[pl.BlockSpec((8, 8192), lambda i: (i, 0))],
        out_specs=pl.BlockSpec((8, 8192), lambda i: (i, 0)),
    )(x)
    return out.reshape(B, height, width, C)


# speedup vs baseline: 7.0549x; 7.0549x over previous
import functools

import jax
import jax.numpy as jnp
from jax import lax
from jax.experimental import pallas as pl
from jax.experimental.pallas import tpu as pltpu
from jax.experimental.pallas import tpu_sc as plsc

B, H, W, C = 4, 112, 112, 192
FI = H * W * C
N = B * FI
OUT = B * (2 * H) * (2 * W) * C
LIVE = 7 * FI

NC, NS = 2, 16
NW = NC * NS
EPT = N // NW
BIN_BITS = 20
CH = 1 << BIN_BITS
NBINS = (LIVE + CH - 1) // CH
CHUNK_PER_TILE = CH // NS
BLK = 512
NBLK = EPT // BLK
CAP = 2048
THRESH = CAP - BLK
MAXF = EPT // THRESH + 1
SEG = MAXF * CAP
TAIL0 = NBINS * CH
TAILW = OUT - TAIL0
TPW = TAILW // NW
ZB = 32768
TFULL, TREM = TPW // ZB, TPW % ZB

_mesh = plsc.VectorSubcoreMesh(core_axis_name="c", subcore_axis_name="s")
_params = pltpu.CompilerParams(
    needs_layout_passes=False, use_tc_tiling_on_sc=False)
_LANE = None


@functools.partial(
    pl.kernel,
    out_type=(
        jax.ShapeDtypeStruct((NW, NBINS, SEG), jnp.int32),
        jax.ShapeDtypeStruct((NW, NBINS, SEG), jnp.float32),
        jax.ShapeDtypeStruct((NW, 32), jnp.int32),
    ),
    mesh=_mesh,
    compiler_params=_params,
    scratch_types=dict(
        sidx=pltpu.VMEM((NBINS, CAP), jnp.int32),
        sval=pltpu.VMEM((NBINS, CAP), jnp.float32),
        idxin=pltpu.VMEM((2, BLK), jnp.int32),
        valin=pltpu.VMEM((2, BLK), jnp.float32),
        cnt=pltpu.VMEM((32,), jnp.int32),
        cvec=pltpu.VMEM((32,), jnp.int32),
        nf=pltpu.SMEM((32,), jnp.int32),
        semi0=pltpu.SemaphoreType.DMA,
        semv0=pltpu.SemaphoreType.DMA,
        semi1=pltpu.SemaphoreType.DMA,
        semv1=pltpu.SemaphoreType.DMA,
    ),
)
def _phase1(mask_h, upd_h, sidx_h, sval_h, cnt_h, sidx, sval, idxin, valin,
            cnt, cvec, nf, semi0, semv0, semi1, semv1):
    cid = lax.axis_index("c")
    sid = lax.axis_index("s")
    wid = cid * NS + sid
    bofs = (wid >> 3) * FI
    ebase = wid * EPT
    lane = lax.iota(jnp.int32, 16)

    for k in range(NBINS):
        def zinit(i, _, k=k):
            sidx[k, pl.ds(i * 16, 16)] = jnp.zeros((16,), jnp.int32)
            sval[k, pl.ds(i * 16, 16)] = jnp.zeros((16,), jnp.float32)
            return 0
        lax.fori_loop(0, CAP // 16, zinit, 0)
    cnt[pl.ds(0, 16)] = jnp.zeros((16,), jnp.int32)
    cnt[pl.ds(16, 16)] = jnp.zeros((16,), jnp.int32)
    for k in range(NBINS):
        nf[k] = 0

    sems = ((semi0, semv0), (semi1, semv1))

    def issue(block, buf):
        base = ebase + block * BLK
        si, sv = sems[buf]
        pltpu.async_copy(mask_h.at[pl.ds(base, BLK)], idxin.at[buf], si)
        pltpu.async_copy(upd_h.at[pl.ds(base, BLK)], valin.at[buf], sv)

    def wait(buf):
        si, sv = sems[buf]
        pltpu.make_async_copy(mask_h.at[pl.ds(0, BLK)], idxin.at[buf], si).wait()
        pltpu.make_async_copy(upd_h.at[pl.ds(0, BLK)], valin.at[buf], sv).wait()

    def process(buf):
        for v in range(BLK // 16):
            idx = idxin[buf, pl.ds(v * 16, 16)] + bofs
            val = valin[buf, pl.ds(v * 16, 16)]
            bin_ = lax.shift_right_logical(idx, BIN_BITS)
            rel = idx & (CH - 1)
            base = plsc.load_gather(cnt, [bin_])
            c, lastm = plsc.scan_count(bin_)
            off = base + c - 1
            plsc.store_scatter(sidx, [bin_, off], rel)
            plsc.store_scatter(sval, [bin_, off], val)
            plsc.addupdate_scatter(cnt, [bin_], c, mask=lastm)
        w0 = cnt[pl.ds(0, 16)]
        w1 = cnt[pl.ds(16, 16)]
        for k in range(NBINS):
            ck = w0[k] if k < 16 else w1[k - 16]

            @pl.when(ck >= THRESH)
            def _flush(k=k):
                nfk = nf[k]
                dst = pl.ds(nfk * CAP, CAP)
                pltpu.sync_copy(sidx.at[k], sidx_h.at[wid, k, dst])
                pltpu.sync_copy(sval.at[k], sval_h.at[wid, k, dst])

                def zero_vals(i, _, k=k):
                    sval[k, pl.ds(i * 16, 16)] = jnp.zeros((16,), jnp.float32)
                    return 0
                lax.fori_loop(0, CAP // 16, zero_vals, 0)
                nf[k] = nfk + 1
                if k < 16:
                    cnt[pl.ds(0, 16)] = jnp.where(
                        lane == k, 0, cnt[pl.ds(0, 16)])
                else:
                    cnt[pl.ds(16, 16)] = jnp.where(
                        lane == k - 16, 0, cnt[pl.ds(16, 16)])

    issue(0, 0)

    def body(j, _):
        b0 = 2 * j
        wait(0)

        @pl.when(b0 + 1 < NBLK)
        def _():
            issue(b0 + 1, 1)
        process(0)
        wait(1)

        @pl.when(b0 + 2 < NBLK)
        def _():
            issue(b0 + 2, 0)
        process(1)
        return 0

    lax.fori_loop(0, NBLK // 2, body, 0)

    w0 = cnt[pl.ds(0, 16)]
    w1 = cnt[pl.ds(16, 16)]
    for k in range(NBINS):
        ck = w0[k] if k < 16 else w1[k - 16]

        @pl.when(ck > 0)
        def _drain(k=k):
            nfk = nf[k]
            dst = pl.ds(nfk * CAP, CAP)
            pltpu.sync_copy(sidx.at[k], sidx_h.at[wid, k, dst])
            pltpu.sync_copy(sval.at[k], sval_h.at[wid, k, dst])
            nf[k] = nfk + 1

    c0 = jnp.zeros((16,), jnp.int32)
    c1 = jnp.zeros((16,), jnp.int32)
    for k in range(NBINS):
        if k < 16:
            c0 = jnp.where(lane == k, nf[k], c0)
        else:
            c1 = jnp.where(lane == k - 16, nf[k], c1)
    cvec[pl.ds(0, 16)] = c0
    cvec[pl.ds(16, 16)] = c1
    pltpu.sync_copy(cvec, cnt_h.at[wid])


@functools.partial(
    pl.kernel,
    out_type=jax.ShapeDtypeStruct((OUT,), jnp.float32),
    mesh=_mesh,
    compiler_params=_params,
    scratch_types=dict(
        shared=pltpu.VMEM_SHARED((CH,), jnp.float32),
        ibuf0=pltpu.VMEM((CAP,), jnp.int32),
        vbuf0=pltpu.VMEM((CAP,), jnp.float32),
        ibuf1=pltpu.VMEM((CAP,), jnp.int32),
        vbuf1=pltpu.VMEM((CAP,), jnp.float32),
        cntv=pltpu.VMEM((32,), jnp.int32),
        zbuf=pltpu.VMEM((ZB,), jnp.float32),
        semi0=pltpu.SemaphoreType.DMA,
        semv0=pltpu.SemaphoreType.DMA,
        semi1=pltpu.SemaphoreType.DMA,
        semv1=pltpu.SemaphoreType.DMA,
    ),
)
def _phase2(sidx_h, sval_h, cnt_h, zero_h, out_h, shared, ibuf0, vbuf0,
            ibuf1, vbuf1, cntv, zbuf, semi0, semv0, semi1, semv1):
    cid = lax.axis_index("c")
    sid = lax.axis_index("s")
    wid = cid * NS + sid
    lane = lax.iota(jnp.int32, 16)

    pltpu.sync_copy(zero_h.at[pl.ds(0, ZB)], zbuf)

    tbase = TAIL0 + wid * TPW

    def tfill(i, _):
        pltpu.sync_copy(zbuf, out_h.at[pl.ds(tbase + i * ZB, ZB)])
        return 0
    lax.fori_loop(0, TFULL, tfill, 0)
    pltpu.sync_copy(zbuf.at[pl.ds(0, TREM)],
                    out_h.at[pl.ds(tbase + TFULL * ZB, TREM)])

    bufs = ((ibuf0, vbuf0, semi0, semv0), (ibuf1, vbuf1, semi1, semv1))

    def seg_scatter(t, q):
        pltpu.sync_copy(cnt_h.at[t], cntv)
        w0 = cntv[pl.ds(0, 16)]
        w1 = cntv[pl.ds(16, 16)]
        nrec = (jnp.max(jnp.where(lane == q, w0, 0), axis=0)
                + jnp.max(jnp.where(lane == q - 16, w1, 0), axis=0))

        def issue(j, b):
            ib, vb, si, sv = bufs[b]
            src = pl.ds(j * CAP, CAP)
            pltpu.async_copy(sidx_h.at[t, q, src], ib, si)
            pltpu.async_copy(sval_h.at[t, q, src], vb, sv)

        def wait_scatter(b):
            ib, vb, si, sv = bufs[b]
            pltpu.make_async_copy(sidx_h.at[0, 0, pl.ds(0, CAP)], ib, si).wait()
            pltpu.make_async_copy(sval_h.at[0, 0, pl.ds(0, CAP)], vb, sv).wait()
            pltpu.sync_copy(vb, shared.at[ib], add=True)

        @pl.when(nrec > 0)
        def _():
            issue(0, 0)

            def body(j, _):
                r0 = 2 * j
                wait_scatter(0)

                @pl.when(r0 + 1 < nrec)
                def _():
                    issue(r0 + 1, 1)

                    @pl.when(r0 + 2 < nrec)
                    def _():
                        issue(r0 + 2, 0)
                    wait_scatter(1)
                return 0

            lax.fori_loop(0, (nrec + 1) >> 1, body, 0)

    for r in range(9):
        q = 2 * r + cid

        @pl.when(q < NBINS)
        def _(q=q):
            zofs = sid * CHUNK_PER_TILE
            pltpu.sync_copy(zero_h.at[pl.ds(zofs, CHUNK_PER_TILE)],
                            shared.at[pl.ds(zofs, CHUNK_PER_TILE)])
            plsc.subcore_barrier()
            seg_scatter(2 * sid, q)
            seg_scatter(2 * sid + 1, q)
            plsc.subcore_barrier()
            pltpu.sync_copy(shared.at[pl.ds(zofs, CHUNK_PER_TILE)],
                            out_h.at[pl.ds(q * CH + zofs, CHUNK_PER_TILE)])
            plsc.subcore_barrier()


def kernel(updates, mask):
    mask_flat = mask.astype(jnp.int32).reshape(-1)
    upd_flat = updates.reshape(-1)
    sidx, sval, cnts = _phase1(mask_flat, upd_flat)
    zeros = jnp.zeros((CH,), jnp.float32)
    out = _phase2(sidx, sval, cnts, zeros)
    return out.reshape(B, 2 * H, 2 * W, C)


# speedup vs baseline: 9.2877x; 1.3165x over previous
import functools

import jax
import jax.numpy as jnp
from jax import lax
from jax.experimental import pallas as pl
from jax.experimental.pallas import tpu as pltpu
from jax.experimental.pallas import tpu_sc as plsc

B, H, W, C = 4, 112, 112, 192
FI = H * W * C
N = B * FI
OUT = B * (2 * H) * (2 * W) * C
LIVE = 7 * FI

NC, NS = 2, 16
NW = NC * NS
EPT = N // NW
BIN_BITS = 20
CH = 1 << BIN_BITS
NBINS = (LIVE + CH - 1) // CH
NVB = 2 * NBINS
CHUNK_PER_TILE = CH // NS
BLK = 512
NBLK = EPT // BLK
REC = 256
NSLOT = 4
ROW = NSLOT * REC
MAXF = (EPT // 2) // REC + 1 + 7
SEG = MAXF * REC
TAIL0 = NBINS * CH
TAILW = OUT - TAIL0
TPW = TAILW // NW
ZB = 16384
TFULL, TREM = TPW // ZB, TPW % ZB
BIG = 1 << 29

_mesh = plsc.VectorSubcoreMesh(core_axis_name="c", subcore_axis_name="s")
_params = pltpu.CompilerParams(
    needs_layout_passes=False, use_tc_tiling_on_sc=False)


@functools.partial(
    pl.kernel,
    out_type=(
        jax.ShapeDtypeStruct((NW, NVB, SEG), jnp.int32),
        jax.ShapeDtypeStruct((NW, NVB, SEG), jnp.float32),
        jax.ShapeDtypeStruct((NW, 48), jnp.int32),
    ),
    mesh=_mesh,
    compiler_params=_params,
    scratch_types=dict(
        sidx=pltpu.VMEM((NVB, ROW), jnp.int32),
        sval=pltpu.VMEM((NVB, ROW), jnp.float32),
        idxin=pltpu.VMEM((2, BLK), jnp.int32),
        valin=pltpu.VMEM((2, BLK), jnp.float32),
        cntA=pltpu.VMEM((32,), jnp.int32),
        cntB=pltpu.VMEM((32,), jnp.int32),
        nxtA=pltpu.VMEM((32,), jnp.int32),
        nxtB=pltpu.VMEM((32,), jnp.int32),
        cvec=pltpu.VMEM((48,), jnp.int32),
        zri=pltpu.VMEM((REC,), jnp.int32),
        zrv=pltpu.VMEM((REC,), jnp.float32),
        nf=pltpu.SMEM((64,), jnp.int32),
        pend=pltpu.SMEM((8,), jnp.int32),
        semi0=pltpu.SemaphoreType.DMA,
        semv0=pltpu.SemaphoreType.DMA,
        semi1=pltpu.SemaphoreType.DMA,
        semv1=pltpu.SemaphoreType.DMA,
        semfi=pltpu.SemaphoreType.DMA,
        semfv=pltpu.SemaphoreType.DMA,
    ),
)
def _phase1(mask_h, upd_h, sidx_h, sval_h, cnt_h, sidx, sval, idxin, valin,
            cntA, cntB, nxtA, nxtB, cvec, zri, zrv, nf, pend,
            semi0, semv0, semi1, semv1, semfi, semfv):
    cid = lax.axis_index("c")
    sid = lax.axis_index("s")
    wid = cid * NS + sid
    bofs = (wid >> 3) * FI
    ebase = wid * EPT
    lane = lax.iota(jnp.int32, 16)

    for k in range(NVB):
        def zinit(i, _, k=k):
            sidx[k, pl.ds(i * 16, 16)] = jnp.zeros((16,), jnp.int32)
            return 0
        lax.fori_loop(0, ROW // 16, zinit, 0)
    for i in range(REC // 16):
        zri[pl.ds(i * 16, 16)] = jnp.zeros((16,), jnp.int32)
        zrv[pl.ds(i * 16, 16)] = jnp.zeros((16,), jnp.float32)
    z16 = jnp.zeros((16,), jnp.int32)
    cntA[pl.ds(0, 16)] = z16
    cntA[pl.ds(16, 16)] = z16
    cntB[pl.ds(0, 16)] = z16
    cntB[pl.ds(16, 16)] = z16
    nxtA[pl.ds(0, 16)] = jnp.full((16,), REC, jnp.int32)
    nxtA[pl.ds(16, 16)] = jnp.where(lane == 0, REC, BIG)
    nxtB[pl.ds(0, 16)] = jnp.full((16,), REC, jnp.int32)
    nxtB[pl.ds(16, 16)] = jnp.where(lane == 0, REC, BIG)
    for k in range(NVB):
        nf[k] = 0
    pend[0] = 0

    sems = ((semi0, semv0), (semi1, semv1))

    def issue(block, buf):
        base = ebase + block * BLK
        si, sv = sems[buf]
        pltpu.async_copy(mask_h.at[pl.ds(base, BLK)], idxin.at[buf], si)
        pltpu.async_copy(upd_h.at[pl.ds(base, BLK)], valin.at[buf], sv)

    def wait(buf):
        si, sv = sems[buf]
        pltpu.make_async_copy(mask_h.at[pl.ds(0, BLK)], idxin.at[buf], si).wait()
        pltpu.make_async_copy(upd_h.at[pl.ds(0, BLK)], valin.at[buf], sv).wait()

    def flush_wait():
        @pl.when(pend[0] > 0)
        def _():
            pltpu.make_async_copy(
                sidx.at[0, pl.ds(0, REC)],
                sidx_h.at[0, 0, pl.ds(0, REC)], semfi).wait()
            pltpu.make_async_copy(
                sval.at[0, pl.ds(0, REC)],
                sval_h.at[0, 0, pl.ds(0, REC)], semfv).wait()
        pend[0] = 1

    def flush_record(k, nfk, src_i, src_v):
        flush_wait()
        dst = pl.ds(nfk * REC, REC)
        pltpu.async_copy(src_i, sidx_h.at[wid, k, dst], semfi)
        pltpu.async_copy(src_v, sval_h.at[wid, k, dst], semfv)
        nf[k] = nfk + 1

    def process(buf):
        for v in range(BLK // 16):
            chain = v & 1
            cnt_c = cntA if chain == 0 else cntB
            idx = idxin[buf, pl.ds(v * 16, 16)] + bofs
            val = valin[buf, pl.ds(v * 16, 16)]
            bin_ = lax.shift_right_logical(idx, BIN_BITS)
            rel = idx & (CH - 1)
            vbin = bin_ * 2 + chain
            base = plsc.load_gather(cnt_c, [bin_])
            c, lastm = plsc.scan_count(bin_)
            tot = base + c
            off = (tot - 1) & (ROW - 1)
            plsc.store_scatter(sidx, [vbin, off], rel)
            plsc.store_scatter(sval, [vbin, off], val)
            plsc.addupdate_scatter(cnt_c, [bin_], c, mask=lastm)
        a0 = cntA[pl.ds(0, 16)]
        a1 = cntA[pl.ds(16, 16)]
        b0 = cntB[pl.ds(0, 16)]
        b1 = cntB[pl.ds(16, 16)]
        d = jnp.maximum(jnp.maximum(a0 - nxtA[pl.ds(0, 16)],
                                    a1 - nxtA[pl.ds(16, 16)]),
                        jnp.maximum(b0 - nxtB[pl.ds(0, 16)],
                                    b1 - nxtB[pl.ds(16, 16)]))
        anyf = jnp.max(d, axis=0)

        @pl.when(anyf >= 0)
        def _scan():
            for k in range(NVB):
                bn, chain = k >> 1, k & 1
                w = (a0, a1) if chain == 0 else (b0, b1)
                ck = w[0][bn] if bn < 16 else w[1][bn - 16]
                nxt = nxtA if chain == 0 else nxtB

                @pl.when(ck >= nf[k] * REC + REC)
                def _flush(k=k, nxt=nxt, bn=bn):
                    nfk = nf[k]
                    slot = nfk & (NSLOT - 1)
                    src = pl.ds(slot * REC, REC)
                    flush_record(k, nfk, sidx.at[k, src], sval.at[k, src])
                    if bn < 16:
                        nxt[pl.ds(0, 16)] = jnp.where(
                            lane == bn, nxt[pl.ds(0, 16)] + REC,
                            nxt[pl.ds(0, 16)])
                    else:
                        nxt[pl.ds(16, 16)] = jnp.where(
                            lane == bn - 16, nxt[pl.ds(16, 16)] + REC,
                            nxt[pl.ds(16, 16)])

    issue(0, 0)

    def body(j, _):
        b0 = 2 * j
        wait(0)

        @pl.when(b0 + 1 < NBLK)
        def _():
            issue(b0 + 1, 1)
        process(0)
        wait(1)

        @pl.when(b0 + 2 < NBLK)
        def _():
            issue(b0 + 2, 0)
        process(1)
        return 0

    lax.fori_loop(0, NBLK // 2, body, 0)

    a0 = cntA[pl.ds(0, 16)]
    a1 = cntA[pl.ds(16, 16)]
    b0 = cntB[pl.ds(0, 16)]
    b1 = cntB[pl.ds(16, 16)]
    for k in range(NVB):
        bn, chain = k >> 1, k & 1
        w = (a0, a1) if chain == 0 else (b0, b1)
        ck = w[0][bn] if bn < 16 else w[1][bn - 16]
        nfk0 = nf[k]
        res = ck - nfk0 * REC

        @pl.when(res > 0)
        def _drain(k=k, nfk0=nfk0, res=res):
            slot = nfk0 & (NSLOT - 1)
            for i in range(REC // 16):
                cur = sval[k, pl.ds(slot * REC + i * 16, 16)]
                sval[k, pl.ds(slot * REC + i * 16, 16)] = jnp.where(
                    lane + (i * 16) < res, cur, 0.0)
            src = pl.ds(slot * REC, REC)
            flush_record(k, nfk0, sidx.at[k, src], sval.at[k, src])

        nfk1 = nf[k]
        npad = (8 - (nfk1 & 7)) & 7

        def pbody(i, _, k=k):
            flush_record(k, nf[k], zri, zrv)
            return 0
        lax.fori_loop(0, npad, pbody, 0)

    @pl.when(pend[0] > 0)
    def _():
        pltpu.make_async_copy(
            sidx.at[0, pl.ds(0, REC)],
            sidx_h.at[0, 0, pl.ds(0, REC)], semfi).wait()
        pltpu.make_async_copy(
            sval.at[0, pl.ds(0, REC)],
            sval_h.at[0, 0, pl.ds(0, REC)], semfv).wait()

    c0 = jnp.zeros((16,), jnp.int32)
    c1 = jnp.zeros((16,), jnp.int32)
    c2 = jnp.zeros((16,), jnp.int32)
    for k in range(NVB):
        if k < 16:
            c0 = jnp.where(lane == k, nf[k], c0)
        elif k < 32:
            c1 = jnp.where(lane == k - 16, nf[k], c1)
        else:
            c2 = jnp.where(lane == k - 32, nf[k], c2)
    cvec[pl.ds(0, 16)] = c0
    cvec[pl.ds(16, 16)] = c1
    cvec[pl.ds(32, 16)] = c2
    pltpu.sync_copy(cvec, cnt_h.at[wid])


RB = 8 * REC


@functools.partial(
    pl.kernel,
    out_type=jax.ShapeDtypeStruct((OUT,), jnp.float32),
    mesh=_mesh,
    compiler_params=_params,
    scratch_types=dict(
        shared=pltpu.VMEM_SHARED((CH,), jnp.float32),
        ibuf0=pltpu.VMEM((RB,), jnp.int32),
        vbuf0=pltpu.VMEM((RB,), jnp.float32),
        ibuf1=pltpu.VMEM((RB,), jnp.int32),
        vbuf1=pltpu.VMEM((RB,), jnp.float32),
        cv0=pltpu.VMEM((48,), jnp.int32),
        cv1=pltpu.VMEM((48,), jnp.int32),
        zbuf=pltpu.VMEM((ZB,), jnp.float32),
        semi0=pltpu.SemaphoreType.DMA,
        semv0=pltpu.SemaphoreType.DMA,
        semi1=pltpu.SemaphoreType.DMA,
        semv1=pltpu.SemaphoreType.DMA,
    ),
)
def _phase2(sidx_h, sval_h, cnt_h, zero_h, out_h, shared, ibuf0, vbuf0,
            ibuf1, vbuf1, cv0, cv1, zbuf, semi0, semv0, semi1, semv1):
    cid = lax.axis_index("c")
    sid = lax.axis_index("s")
    wid = cid * NS + sid
    lane = lax.iota(jnp.int32, 16)

    pltpu.sync_copy(zero_h.at[pl.ds(0, ZB)], zbuf)

    tbase = TAIL0 + wid * TPW

    def tfill(i, _):
        pltpu.sync_copy(zbuf, out_h.at[pl.ds(tbase + i * ZB, ZB)])
        return 0
    lax.fori_loop(0, TFULL, tfill, 0)
    pltpu.sync_copy(zbuf.at[pl.ds(0, TREM)],
                    out_h.at[pl.ds(tbase + TFULL * ZB, TREM)])

    pltpu.sync_copy(cnt_h.at[2 * sid], cv0)
    pltpu.sync_copy(cnt_h.at[2 * sid + 1], cv1)

    bufs = ((ibuf0, vbuf0, semi0, semv0), (ibuf1, vbuf1, semi1, semv1))

    def seg_scatter(t, tt, vb):
        cv = cv0 if tt == 0 else cv1
        w0 = cv[pl.ds(0, 16)]
        w1 = cv[pl.ds(16, 16)]
        w2 = cv[pl.ds(32, 16)]
        nrec = (jnp.max(jnp.where(lane == vb, w0, 0), axis=0)
                + jnp.max(jnp.where(lane == vb - 16, w1, 0), axis=0)
                + jnp.max(jnp.where(lane == vb - 32, w2, 0), axis=0))
        nb = lax.shift_right_logical(nrec, 3)

        def issue(j, b):
            ib, vbuf, si, sv = bufs[b]
            src = pl.ds(j * RB, RB)
            pltpu.async_copy(sidx_h.at[t, vb, src], ib, si)
            pltpu.async_copy(sval_h.at[t, vb, src], vbuf, sv)

        def wait_scatter(b):
            ib, vbuf, si, sv = bufs[b]
            pltpu.make_async_copy(
                sidx_h.at[0, 0, pl.ds(0, RB)], ib, si).wait()
            pltpu.make_async_copy(
                sval_h.at[0, 0, pl.ds(0, RB)], vbuf, sv).wait()
            pltpu.sync_copy(vbuf, shared.at[ib], add=True)

        @pl.when(nb > 0)
        def _():
            issue(0, 0)

            def body(j, _):
                r0 = 2 * j
                wait_scatter(0)

                @pl.when(r0 + 1 < nb)
                def _():
                    issue(r0 + 1, 1)

                    @pl.when(r0 + 2 < nb)
                    def _():
                        issue(r0 + 2, 0)
                    wait_scatter(1)
                return 0

            lax.fori_loop(0, (nb + 1) >> 1, body, 0)

    for r in range(9):
        q = 2 * r + cid

        @pl.when(q < NBINS)
        def _(q=q):
            zofs = sid * CHUNK_PER_TILE
            pltpu.sync_copy(zero_h.at[pl.ds(zofs, CHUNK_PER_TILE)],
                            shared.at[pl.ds(zofs, CHUNK_PER_TILE)])
            plsc.subcore_barrier()
            seg_scatter(2 * sid, 0, 2 * q)
            seg_scatter(2 * sid, 0, 2 * q + 1)
            seg_scatter(2 * sid + 1, 1, 2 * q)
            seg_scatter(2 * sid + 1, 1, 2 * q + 1)
            plsc.subcore_barrier()
            pltpu.sync_copy(shared.at[pl.ds(zofs, CHUNK_PER_TILE)],
                            out_h.at[pl.ds(q * CH + zofs, CHUNK_PER_TILE)])


def kernel(updates, mask):
    mask_flat = mask.astype(jnp.int32).reshape(-1)
    upd_flat = updates.reshape(-1)
    sidx, sval, cnts = _phase1(mask_flat, upd_flat)
    zeros = jnp.zeros((CH,), jnp.float32)
    out = _phase2(sidx, sval, cnts, zeros)
    return out.reshape(B, 2 * H, 2 * W, C)
